# baseline (speedup 1.0000x reference)
import jax
import jax.numpy as jnp
from jax import lax
from jax.experimental import pallas as pl
from jax.experimental.pallas import tpu as pltpu
from jax.experimental.pallas import tpu_sc as plsc

VOCAB = 1000000
MAX_LEN = 200
FEAT = 64
BATCH = 4096

NC = 2
NS = 16
NW = NC * NS
BB = BATCH // NW
LANES = 16
NBB = BB // LANES
VPF = FEAT // LANES
FT = FEAT // 8
NBUF = 4
PITCH = 65


def _emb_body(x_hbm, emb_hbm, pe_hbm, out_hbm,
              xs, pe_v, idxb, rows, rk, otile, gsem, osem):
    B = lax.axis_index("s") * NC + lax.axis_index("c")
    pltpu.sync_copy(x_hbm.at[pl.ds(B * BB, BB)], xs)
    pltpu.sync_copy(pe_hbm, pe_v)

    iota = lax.iota(jnp.int32, LANES)

    def issue_gather(pos, b):
        col = jnp.full((LANES,), pos, jnp.int32)
        for j in range(NBB):
            v = plsc.load_gather(xs, [iota + (j * LANES), col])
            idxb[b, pl.ds(j * LANES, LANES)] = v
        pltpu.async_copy(emb_hbm.at[idxb.at[b]], rows.at[b], gsem.at[b])

    def wait_gather(b):
        pltpu.make_async_copy(emb_hbm.at[idxb.at[b]], rows.at[b],
                              gsem.at[b]).wait()

    def issue_store(l, b):
        for F in range(FT):
            pltpu.async_copy(otile.at[b, pl.ds(F * 8, 8)],
                             out_hbm.at[l, F, B], osem.at[b])

    def wait_store(b):
        for F in range(FT):
            pltpu.make_async_copy(otile.at[b, pl.ds(F * 8, 8)],
                                  out_hbm.at[0, F, B], osem.at[b]).wait()

    def transpose_add(l, b):
        pes = [pe_v[l, pl.ds(c * LANES, LANES)] for c in range(VPF)]

        def jbody(j, _):
            for u in range(2):
                jj = 2 * j + u
                for c in range(VPF):
                    v = rows[b, jj, pl.ds(c * LANES, LANES)] + pes[c]
                    plsc.store_scatter(rk, [jj * PITCH + (c * LANES) + iota], v)
            return 0

        lax.fori_loop(0, BB // 2, jbody, 0)

        cbase = [(iota + m * LANES) * PITCH for m in range(NBB)]

        def fbody(f2, _):
            for u in range(2):
                f = 2 * f2 + u
                fb = jnp.full((LANES,), f, jnp.int32)
                for m in range(NBB):
                    col = plsc.load_gather(rk, [cbase[m] + fb])
                    otile[b, f, pl.ds(m * LANES, LANES)] = col
            return 0

        lax.fori_loop(0, FEAT // 2, fbody, 0)

    for p in range(NBUF - 1):
        issue_gather(p, p)

    def outer(g, _):
        for b in range(NBUF):
            l = NBUF * g + b
            bf = (b + NBUF - 1) % NBUF
            fpos = l + NBUF - 1

            @pl.when(fpos < MAX_LEN)
            def _():
                issue_gather(fpos, bf)

            wait_gather(b)

            @pl.when(l >= NBUF)
            def _():
                wait_store(b)

            transpose_add(l, b)
            issue_store(l, b)
        return 0

    lax.fori_loop(0, MAX_LEN // NBUF, outer, 0)
    for b in range(NBUF):
        wait_store(b)


def kernel(x, emb_token, pe):
    x = x.astype(jnp.int32)
    mesh = plsc.VectorSubcoreMesh(core_axis_name="c", subcore_axis_name="s")
    out5 = pl.kernel(
        _emb_body,
        out_type=jax.ShapeDtypeStruct((MAX_LEN, FT, NW, 8, 128), jnp.float32),
        mesh=mesh,
        compiler_params=pltpu.CompilerParams(use_tc_tiling_on_sc=False,
                                             needs_layout_passes=False),
        scratch_types=[
            pltpu.VMEM((BB, MAX_LEN), jnp.int32),
            pltpu.VMEM((MAX_LEN, FEAT), jnp.float32),
            pltpu.VMEM((NBUF, BB), jnp.int32),
            pltpu.VMEM((NBUF, BB, FEAT), jnp.float32),
            pltpu.VMEM((BB * PITCH,), jnp.float32),
            pltpu.VMEM((NBUF, FEAT, 128), jnp.float32),
            pltpu.SemaphoreType.DMA((NBUF,)),
            pltpu.SemaphoreType.DMA((NBUF,)),
        ],
    )(x, emb_token, pe)
    return out5.transpose(2, 4, 0, 1, 3).reshape(BATCH, MAX_LEN, FEAT)


# speedup vs baseline: 1.1864x; 1.1864x over previous
import jax
import jax.numpy as jnp
from jax import lax
from jax.experimental import pallas as pl
from jax.experimental.pallas import tpu as pltpu
from jax.experimental.pallas import tpu_sc as plsc

VOCAB = 1000000
MAX_LEN = 200
FEAT = 64
BATCH = 4096

NC = 2
NS = 16
NW = NC * NS
SEQ_PER_W = BATCH // NW
LANES = 16
VPF = FEAT // LANES
NBUF = 4
CH0, CH1 = 128, MAX_LEN - 128


def _emb_body(x_hbm, emb_hbm, pe_hbm, out_hbm, idx_all, rows, pe_v, gsem, osem):
    wid = lax.axis_index("s") * NC + lax.axis_index("c")
    base = wid * SEQ_PER_W
    pltpu.sync_copy(pe_hbm, pe_v)
    pltpu.sync_copy(x_hbm.at[pl.ds(base, SEQ_PER_W)], idx_all)

    def issue_gather(s, b):
        pltpu.async_copy(emb_hbm.at[idx_all.at[s, pl.ds(0, CH0)]],
                         rows.at[b, pl.ds(0, CH0)], gsem.at[b])
        pltpu.async_copy(emb_hbm.at[idx_all.at[s, pl.ds(CH0, CH1)]],
                         rows.at[b, pl.ds(CH0, CH1)], gsem.at[b])

    def wait_gather(s, b):
        pltpu.make_async_copy(emb_hbm.at[idx_all.at[s, pl.ds(0, CH0)]],
                              rows.at[b, pl.ds(0, CH0)], gsem.at[b]).wait()
        pltpu.make_async_copy(emb_hbm.at[idx_all.at[s, pl.ds(CH0, CH1)]],
                              rows.at[b, pl.ds(CH0, CH1)], gsem.at[b]).wait()

    def wait_store(b):
        pltpu.make_async_copy(rows.at[b], out_hbm.at[base], osem.at[b]).wait()

    for b in range(NBUF - 1):
        issue_gather(b, b)

    def outer(g, _):
        for b in range(NBUF):
            s = NBUF * g + b
            bf = (b + NBUF - 1) % NBUF
            f = s + NBUF - 1

            @pl.when(jnp.logical_and(s >= 1, f < SEQ_PER_W))
            def _():
                wait_store(bf)

            @pl.when(f < SEQ_PER_W)
            def _():
                issue_gather(f, bf)

            wait_gather(s, b)

            def add_rows(r, _):
                for u in range(2):
                    rr = 2 * r + u
                    for c in range(VPF):
                        sl = pl.ds(c * LANES, LANES)
                        rows[b, rr, sl] = rows[b, rr, sl] + pe_v[rr, sl]
                return 0

            lax.fori_loop(0, MAX_LEN // 2, add_rows, 0)
            pltpu.async_copy(rows.at[b], out_hbm.at[base + s], osem.at[b])
        return 0

    lax.fori_loop(0, SEQ_PER_W // NBUF, outer, 0)
    for b in range(NBUF):
        wait_store(b)


def kernel(x, emb_token, pe):
    x = x.astype(jnp.int32)
    mesh = plsc.VectorSubcoreMesh(core_axis_name="c", subcore_axis_name="s")
    return pl.kernel(
        _emb_body,
        out_type=jax.ShapeDtypeStruct((BATCH, MAX_LEN, FEAT), jnp.float32),
        mesh=mesh,
        compiler_params=pltpu.CompilerParams(use_tc_tiling_on_sc=False),
        scratch_types=[
            pltpu.VMEM((SEQ_PER_W, MAX_LEN), jnp.int32),
            pltpu.VMEM((NBUF, MAX_LEN, FEAT), jnp.float32),
            pltpu.VMEM((MAX_LEN, FEAT), jnp.float32),
            pltpu.SemaphoreType.DMA((NBUF,)),
            pltpu.SemaphoreType.DMA((NBUF,)),
        ],
    )(x, emb_token, pe)
